# baseline (speedup 1.0000x reference)
import functools

import jax
import jax.numpy as jnp
from jax import lax
from jax.experimental import pallas as pl
from jax.experimental.pallas import tpu as pltpu
from jax.experimental.pallas import tpu_sc as plsc

_N_DATA = 1000000
_N_CLASSES = 100
_BATCH = 16384

_NC, _NS = 2, 16
_NW = _NC * _NS
_BPW = _BATCH // _NW
_CHUNK = 128
_NCHUNK = _BPW // _CHUNK

_SC_MESH = plsc.VectorSubcoreMesh(core_axis_name="c", subcore_axis_name="s")


_GCHUNK = 64
_NGCHUNK = _BPW // _GCHUNK


_LAG = 16


def _gather_body(conf3_hbm, gidx_hbm, r8_hbm, out_hbm,
                 gidx_v, r8_v, rows_v, sem):
    wid = lax.axis_index("s") * _NC + lax.axis_index("c")
    base = wid * _BPW
    pltpu.sync_copy(gidx_hbm.at[pl.ds(wid * _NCHUNK, _NCHUNK)], gidx_v)
    pltpu.sync_copy(r8_hbm.at[pl.ds(wid * _NCHUNK, _NCHUNK)], r8_v)
    lanes = lax.iota(jnp.int32, 16)

    def drain(p):
        pltpu.make_async_copy(
            conf3_hbm.at[0, 0], rows_v.at[p, pl.ds(0, _N_CLASSES)],
            sem).wait()

    def vec_body(q, _):
        gv = gidx_v[q // 8, pl.ds((q % 8) * 16, 16)]
        rv = r8_v[q // 8, pl.ds((q % 8) * 16, 16)]
        for l in range(16):
            p = q * 16 + l
            sel = lanes == l
            g = jnp.max(jnp.where(sel, gv, 0))
            r8 = jnp.max(jnp.where(sel, rv, 0))

            @pl.when(p >= _LAG)
            def _():
                drain(p - _LAG)
            pltpu.async_copy(
                conf3_hbm.at[g, r8], rows_v.at[p, pl.ds(0, _N_CLASSES)],
                sem)
        return 0

    lax.fori_loop(0, _BPW // 16, vec_body, 0)

    def drain_body(p, _):
        drain(p)
        return 0
    lax.fori_loop(_BPW - _LAG, _BPW, drain_body, 0)

    pltpu.sync_copy(rows_v, out_hbm.at[pl.ds(base, _BPW)])


def _sc_gather(confidence, gidx2d, r82d):
    conf3 = confidence.reshape(_N_DATA // 8, 8, _N_CLASSES)
    return pl.kernel(
        _gather_body,
        out_type=jax.ShapeDtypeStruct((_BATCH, 128), jnp.float32),
        mesh=_SC_MESH,
        scratch_types=[
            pltpu.VMEM((_NCHUNK, _CHUNK), jnp.int32),
            pltpu.VMEM((_NCHUNK, _CHUNK), jnp.int32),
            pltpu.VMEM((_BPW, 128), jnp.float32),
            pltpu.SemaphoreType.DMA,
        ],
        compiler_params=pltpu.CompilerParams(needs_layout_passes=False),
    )(conf3, gidx2d, r82d)

_ROWS_PER_BLOCK = 2048
_N_BLOCKS = _BATCH // _ROWS_PER_BLOCK


def _compute_body(o_ref, t_ref, nt_ref, loss_ref):
    pid = pl.program_id(0)

    x = o_ref[...]
    t = t_ref[...]
    m = jnp.max(x, axis=1, keepdims=True)
    e = jnp.exp(x - m)
    s = jnp.sum(e, axis=1, keepdims=True)
    p = e / s
    logp = (x - m) - jnp.log(s)
    block_loss = jnp.sum(t * logp)

    r = jnp.where(t > 0, p, jnp.zeros_like(p))
    nt = r / jnp.sum(r, axis=1, keepdims=True)
    nt_ref[...] = nt

    @pl.when(pid == 0)
    def _():
        loss_ref[0, 0] = 0.0

    loss_ref[0, 0] += -block_loss / _BATCH


def _compute_tc(output1, target):
    nt, loss = pl.pallas_call(
        _compute_body,
        grid=(_N_BLOCKS,),
        in_specs=[
            pl.BlockSpec((_ROWS_PER_BLOCK, _N_CLASSES), lambda i: (i, 0)),
            pl.BlockSpec((_ROWS_PER_BLOCK, _N_CLASSES), lambda i: (i, 0)),
        ],
        out_specs=[
            pl.BlockSpec((_ROWS_PER_BLOCK, _N_CLASSES), lambda i: (i, 0)),
            pl.BlockSpec(memory_space=pltpu.SMEM, block_shape=(1, 1),
                         index_map=lambda i: (0, 0)),
        ],
        out_shape=[
            jax.ShapeDtypeStruct((_BATCH, _N_CLASSES), jnp.float32),
            jax.ShapeDtypeStruct((1, 1), jnp.float32),
        ],
    )(output1, target)
    return loss[0, 0], nt


def kernel(output1, index, confidence):
    gidx2d = (index // 8).reshape(_BATCH // _CHUNK, _CHUNK)
    r82d = (index % 8).reshape(_BATCH // _CHUNK, _CHUNK)
    target = _sc_gather(confidence, gidx2d, r82d)[:, :_N_CLASSES]
    loss, new_target = _compute_tc(output1, target)

    order = jnp.argsort(index, stable=True)
    si = index[order]
    is_last = jnp.concatenate(
        [si[1:] != si[:-1], jnp.ones((1,), dtype=bool)])
    live = jnp.zeros((_BATCH,), dtype=bool).at[order].set(is_last)
    idx2 = jnp.where(live, index, _N_DATA)
    new_confidence = confidence.at[idx2].set(new_target, mode="drop")
    return loss, new_confidence
